# baseline (speedup 1.0000x reference)
import functools

import jax
import jax.numpy as jnp
from jax import lax
from jax.experimental import pallas as pl
from jax.experimental.pallas import tpu as pltpu
from jax.experimental.pallas import tpu_sc as plsc

_NUM_SC = 2
_NUM_TEC = 16
_NW = _NUM_SC * _NUM_TEC

_VB = 1024


def _make_sc_gather(V, D, B):
    b_per_w = B // _NW
    mesh = plsc.VectorSubcoreMesh(core_axis_name="c", subcore_axis_name="s")

    @functools.partial(
        pl.kernel,
        mesh=mesh,
        out_type=jax.ShapeDtypeStruct((B, D), jnp.float32),
        scratch_types=[
            pltpu.VMEM((b_per_w,), jnp.int32),
            pltpu.VMEM((b_per_w, D), jnp.float32),
            pltpu.SemaphoreType.DMA,
        ],
        compiler_params=pltpu.CompilerParams(use_tc_tiling_on_sc=False),
    )
    def gather_kernel(table_hbm, idx_hbm, out_hbm, idx_v, rows_v, sem):
        wid = lax.axis_index("s") * _NUM_SC + lax.axis_index("c")
        base = wid * b_per_w
        pltpu.sync_copy(idx_hbm.at[pl.ds(base, b_per_w)], idx_v)
        pltpu.async_copy(table_hbm.at[idx_v], rows_v, sem).wait()
        pltpu.sync_copy(rows_v, out_hbm.at[pl.ds(base, b_per_w)])

    return gather_kernel


def _proj_body(cv_ref, w_ref, b_ref, out_ref):
    w_aug = jnp.concatenate([w_ref[...], b_ref[...]], axis=0)
    out_ref[...] = lax.dot_general(
        w_aug,
        cv_ref[...],
        (((0,), (0,)), ((), ())),
        preferred_element_type=jnp.float32,
    )


def kernel(center, emb_table, W, b):
    V, D = emb_table.shape
    B = center.shape[0]

    cv = _make_sc_gather(V, D, B)(emb_table, center.astype(jnp.int32))

    cv_t_aug = jnp.concatenate([cv.T, jnp.ones((1, B), jnp.float32)], axis=0)

    nblk = (V + _VB - 1) // _VB
    out_t = pl.pallas_call(
        _proj_body,
        grid=(nblk,),
        in_specs=[
            pl.BlockSpec((D + 1, B), lambda i: (0, 0)),
            pl.BlockSpec((D, _VB), lambda i: (0, i)),
            pl.BlockSpec((1, _VB), lambda i: (0, i)),
        ],
        out_specs=pl.BlockSpec((_VB, B), lambda i: (i, 0)),
        out_shape=jax.ShapeDtypeStruct((V, B), jnp.float32),
    )(cv_t_aug, W.T, b[None, :])
    return out_t.T


# speedup vs baseline: 1.0889x; 1.0889x over previous
import functools

import jax
import jax.numpy as jnp
from jax import lax
from jax.experimental import pallas as pl
from jax.experimental.pallas import tpu as pltpu
from jax.experimental.pallas import tpu_sc as plsc

_NUM_SC = 2
_NUM_TEC = 16
_NW = _NUM_SC * _NUM_TEC

_VB = 2048


def _make_sc_gather(V, D, B):
    b_per_w = B // _NW
    mesh = plsc.VectorSubcoreMesh(core_axis_name="c", subcore_axis_name="s")

    @functools.partial(
        pl.kernel,
        mesh=mesh,
        out_type=jax.ShapeDtypeStruct((B, D), jnp.float32),
        scratch_types=[
            pltpu.VMEM((b_per_w,), jnp.int32),
            pltpu.VMEM((b_per_w, D), jnp.float32),
            pltpu.SemaphoreType.DMA,
        ],
        compiler_params=pltpu.CompilerParams(use_tc_tiling_on_sc=False),
    )
    def gather_kernel(table_hbm, idx_hbm, out_hbm, idx_v, rows_v, sem):
        wid = lax.axis_index("s") * _NUM_SC + lax.axis_index("c")
        base = wid * b_per_w
        pltpu.sync_copy(idx_hbm.at[pl.ds(base, b_per_w)], idx_v)
        pltpu.async_copy(table_hbm.at[idx_v], rows_v, sem).wait()
        pltpu.sync_copy(rows_v, out_hbm.at[pl.ds(base, b_per_w)])

    return gather_kernel


def _proj_body(cv_ref, w_ref, b_ref, out_ref):
    w_aug = jnp.concatenate([w_ref[...], b_ref[...]], axis=0)
    out_ref[...] = lax.dot_general(
        w_aug,
        cv_ref[...],
        (((0,), (0,)), ((), ())),
        preferred_element_type=jnp.float32,
    )


def kernel(center, emb_table, W, b):
    V, D = emb_table.shape
    B = center.shape[0]

    cv = _make_sc_gather(V, D, B)(emb_table, center.astype(jnp.int32))

    cv_t_aug = jnp.concatenate([cv.T, jnp.ones((1, B), jnp.float32)], axis=0)

    nblk = (V + _VB - 1) // _VB
    out_t = pl.pallas_call(
        _proj_body,
        grid=(nblk,),
        in_specs=[
            pl.BlockSpec((D + 1, B), lambda i: (0, 0)),
            pl.BlockSpec((D, _VB), lambda i: (0, i)),
            pl.BlockSpec((1, _VB), lambda i: (0, i)),
        ],
        out_specs=pl.BlockSpec((_VB, B), lambda i: (i, 0)),
        out_shape=jax.ShapeDtypeStruct((V, B), jnp.float32),
    )(cv_t_aug, W.T, b[None, :])
    return out_t.T
